# Initial kernel scaffold; baseline (speedup 1.0000x reference)
import jax
import jax.numpy as jnp
from jax.experimental import pallas as pl


def kernel(query, knowledge_embed):
    raise NotImplementedError("write your pallas kernel here")


# speedup vs baseline: 22.7910x; 22.7910x over previous
import functools

import jax
import jax.numpy as jnp
from jax import lax
from jax.experimental import pallas as pl
from jax.experimental.pallas import tpu as pltpu
from jax.experimental.pallas import tpu_sc as plsc

K_TOTAL = 1_000_000
Q = 64
D = 32
KB = 16384
NBLK = (K_TOTAL + KB - 1) // KB
NPAD = NBLK * KB
NCOL = NPAD // 128
TOPK_N = 32

NC, NS, L = 2, 16, 16
NW = NC * NS
QPW = Q // NW
WINC = 128
NCT = NCOL // 128
GCAP = 128
CAP = 1024
INT_MAX = 2**31 - 1


def _tc_body(qt_ref, kt_ref, sims_ref, thr_ref, cmax_ref,
             bmax_ref, cm_ref):
    j = pl.program_id(0)
    sims = lax.dot_general(
        qt_ref[...], kt_ref[...], (((0,), (0,)), ((), ())),
        preferred_element_type=jnp.float32)
    gidx = j * KB + lax.broadcasted_iota(jnp.int32, (Q, KB), 1)
    sims = jnp.where(gidx < K_TOTAL, sims, -jnp.inf)

    mcols = []
    for c in range(WINC):
        vc = sims[:, c * 128:(c + 1) * 128]
        sims_ref[:, c, :, :] = vc.reshape(8, 8, 128)
        mcols.append(jnp.max(vc, axis=1, keepdims=True))
    colmax = jnp.concatenate(mcols, axis=1)

    @pl.when(j == 0)
    def _():
        bmax_ref[...] = jnp.full((Q, 128), -jnp.inf, jnp.float32)

    lane = lax.broadcasted_iota(jnp.int32, (Q, 128), 1)
    bmax = jnp.max(colmax, axis=1, keepdims=True)
    bmax_ref[...] = jnp.where(lane == j, bmax, bmax_ref[...])

    cm_ref[:, pl.ds(j * 128, 128)] = colmax

    @pl.when(j == NBLK - 1)
    def _():
        cmax_ref[...] = jnp.transpose(
            cm_ref[...].reshape(8, 8, NCT, 128), (0, 2, 1, 3))

        def step(i, mm):
            rmax = jnp.max(mm, axis=1, keepdims=True)
            return jnp.where(mm == rmax, -jnp.inf, mm)
        mm = lax.fori_loop(0, TOPK_N - 1, step, bmax_ref[...])
        t = jnp.max(mm, axis=1, keepdims=True)
        thr_ref[...] = jnp.broadcast_to(t, (Q, 128))


def _tc_stage(query_t, knowledge_t):
    return pl.pallas_call(
        _tc_body,
        grid=(NBLK,),
        in_specs=[
            pl.BlockSpec((D, Q), lambda j: (0, 0)),
            pl.BlockSpec((D, KB), lambda j: (0, j)),
        ],
        out_specs=[
            pl.BlockSpec((8, WINC, 8, 128), lambda j: (0, j, 0, 0)),
            pl.BlockSpec((Q, 128), lambda j: (0, 0)),
            pl.BlockSpec((8, NCT, 8, 128), lambda j: (0, 0, 0, 0)),
        ],
        out_shape=[
            jax.ShapeDtypeStruct((8, NCOL, 8, 128), jnp.float32),
            jax.ShapeDtypeStruct((Q, 128), jnp.float32),
            jax.ShapeDtypeStruct((8, NCT, 8, 128), jnp.float32),
        ],
        scratch_shapes=[
            pltpu.VMEM((Q, 128), jnp.float32),
            pltpu.VMEM((Q, NCT * 128), jnp.float32),
        ],
    )(query_t, knowledge_t)


def _sc_body(sims2_hbm, thr_hbm, cmax_hbm, out_hbm,
             cm_v, gath_v, cval_v, cidx_v, thr_v, colb_v, rowid_v, orow_v,
             sem):
    wid = lax.axis_index("s") * NC + lax.axis_index("c")
    lanes = lax.iota(jnp.int32, L)
    minus_inf = jnp.full((L,), -jnp.inf, jnp.float32)

    def do_query(qi, _):
        q = wid * QPW + qi
        qg = q // 8
        r = q % 8
        pltpu.sync_copy(thr_hbm.at[q, pl.ds(0, L)], thr_v)
        pltpu.sync_copy(cmax_hbm.at[qg, :, r, :], cm_v)
        t = thr_v[...]

        def init(i, _):
            cval_v[pl.ds(i * L, L)] = minus_inf
            cidx_v[pl.ds(i * L, L)] = jnp.full((L,), INT_MAX, jnp.int32)
            return 0
        lax.fori_loop(0, (CAP + 64) // L, init, 0)

        def col_group(cc, nc):
            m_any = cm_v[cc, pl.ds(0, L)] >= t
            for u in range(1, 8):
                m_any = m_any | (cm_v[cc, pl.ds(u * L, L)] >= t)

            def slow(n):
                for u in range(8):
                    ids = lax.broadcast(cc * 128 + u * L, (L,)) + lanes
                    m = (cm_v[cc, pl.ds(u * L, L)] >= t) & (ids < NCOL)
                    plsc.store_compressed(colb_v.at[pl.ds(n, L)], ids, mask=m)
                    cnt = plsc.all_reduce_population_count(m)
                    cnt = cnt if cnt.ndim == 0 else jnp.max(cnt, axis=0)
                    n = jnp.minimum(n + cnt, GCAP - L)
                return n

            return lax.cond(jnp.any(m_any), slow, lambda n: n, nc)

        nc = lax.fori_loop(0, NCT, col_group, 0)

        for v in range(GCAP // L):
            slot = lax.broadcast(v * L, (L,)) + lanes
            col = jnp.where(slot < nc, colb_v[pl.ds(v * L, L)], slot)
            colb_v[pl.ds(v * L, L)] = col
            rowid_v[pl.ds(v * L, L)] = (qg * NCOL + col) * 8 + r
        pltpu.async_copy(sims2_hbm.at[rowid_v], gath_v, sem).wait()

        def scan_col(c, ptr):
            cvec = colb_v[pl.ds((c // L) * L, L)]
            colbase = jnp.take_along_axis(
                cvec, lax.broadcast(c % L, (L,)), axis=0) * 128

            def do_col(p):
                for u in range(8):
                    v = gath_v[c, pl.ds(u * L, L)]
                    m = v >= t
                    gv = colbase + u * L + lanes
                    plsc.store_compressed(cval_v.at[pl.ds(p, L)], v, mask=m)
                    plsc.store_compressed(cidx_v.at[pl.ds(p, L)], gv, mask=m)
                    cnt = plsc.all_reduce_population_count(m)
                    cnt = cnt if cnt.ndim == 0 else jnp.max(cnt, axis=0)
                    p = jnp.minimum(p + cnt, CAP)
                return p

            m_any = gath_v[c, pl.ds(0, L)] >= t
            for u in range(1, 8):
                m_any = m_any | (gath_v[c, pl.ds(u * L, L)] >= t)
            return lax.cond(jnp.any(m_any), do_col, lambda p: p, ptr)

        ptr = lax.fori_loop(0, nc, scan_col, 0)
        nv = (ptr + L - 1) // L

        def extract(k, _):
            def red1(i, acc):
                return jnp.maximum(acc, cval_v[pl.ds(i * L, L)])
            mx = lax.fori_loop(0, nv, red1, minus_inf)
            m = jnp.max(mx, axis=0)

            def red2(i, acc):
                v = cval_v[pl.ds(i * L, L)]
                ix = cidx_v[pl.ds(i * L, L)]
                return jnp.minimum(acc, jnp.where(v == m, ix, INT_MAX))
            mi = lax.fori_loop(
                0, nv, red2, jnp.full((L,), INT_MAX, jnp.int32))
            gi = jnp.min(mi, axis=0)

            def kill(i, _):
                v = cval_v[pl.ds(i * L, L)]
                ix = cidx_v[pl.ds(i * L, L)]
                cval_v[pl.ds(i * L, L)] = jnp.where(
                    (v == m) & (ix == gi), minus_inf, v)
                return 0
            lax.fori_loop(0, nv, kill, 0)

            plsc.store_scatter(
                orow_v, [lax.broadcast(k, (L,))], lax.broadcast(gi, (L,)),
                mask=lanes == 0)
            return 0
        lax.fori_loop(0, TOPK_N, extract, 0)

        pltpu.sync_copy(orow_v.at[pl.ds(0, TOPK_N)], out_hbm.at[q])
        return 0

    lax.fori_loop(0, QPW, do_query, 0)


@functools.cache
def _sc_topk():
    mesh = plsc.VectorSubcoreMesh(
        core_axis_name="c", subcore_axis_name="s",
        num_cores=NC, num_subcores=NS)
    return pl.kernel(
        _sc_body,
        out_type=jax.ShapeDtypeStruct((Q, TOPK_N), jnp.int32),
        name="sc_filter_topk",
        mesh=mesh,
        compiler_params=pltpu.CompilerParams(
            needs_layout_passes=False, use_tc_tiling_on_sc=False),
        scratch_types=[
            pltpu.VMEM((NCT, 128), jnp.float32),
            pltpu.VMEM((GCAP, 128), jnp.float32),
            pltpu.VMEM((CAP + 64,), jnp.float32),
            pltpu.VMEM((CAP + 64,), jnp.int32),
            pltpu.VMEM((L,), jnp.float32),
            pltpu.VMEM((GCAP,), jnp.int32),
            pltpu.VMEM((GCAP,), jnp.int32),
            pltpu.VMEM((48,), jnp.int32),
            pltpu.SemaphoreType.DMA,
        ],
    )


@jax.jit
def kernel(query, knowledge_embed):
    sims4, thr, cmax4 = _tc_stage(query.T, knowledge_embed.T)
    sims2 = sims4.reshape(8 * NCOL * 8, 128)
    return _sc_topk()(sims2, thr, cmax4)
